# Initial kernel scaffold; baseline (speedup 1.0000x reference)
import jax
import jax.numpy as jnp
from jax.experimental import pallas as pl


def kernel(h, x, edge_index, W1, b1, W2, b2, W3, b3, W4, b4, W5, b5):
    raise NotImplementedError("write your pallas kernel here")


# speedup vs baseline: 3.9149x; 3.9149x over previous
import jax
import jax.numpy as jnp
from jax import lax
from jax.experimental import pallas as pl
from jax.experimental.pallas import tpu as pltpu
from jax.experimental.pallas import tpu_sc as plsc

F32 = jnp.float32

_NC, _NS = 2, 16
_NW = _NC * _NS
_SUB = 80
_NSUB = 5
_CH = _SUB * _NSUB
_EW = 10000
_CPW = _EW // _CH
_NPT = 640
_NPAD = _NS * _NPT
_SCH = 8 * _SUB
_NSUP = 320000 // _SCH
_NSUP_LO = _NSUP // _NW
_NSUP_HI = _NSUP_LO + 1
_NSUP_EXTRA = _NSUP - _NSUP_LO * _NW
_XR = 80


def _silu(v):
    return v * jax.nn.sigmoid(v)


def _sc_gather1_body(tbl_hbm, idx_hbm, out_hbm, buf_v, idx_v, sem):
    c = lax.axis_index("c")
    s = lax.axis_index("s")
    wid = c * _NS + s

    def chunk(j, carry):
        ebase = wid * _EW + j * _CH
        pltpu.sync_copy(idx_hbm.at[pl.ds(ebase, _CH)], idx_v)
        da = []
        for k in range(_NSUB):
            sl = pl.ds(_SUB * k, _SUB)
            da.append(pltpu.async_copy(tbl_hbm.at[idx_v.at[sl]], buf_v.at[sl],
                                       sem))
        for d in da:
            d.wait()
        pltpu.sync_copy(buf_v, out_hbm.at[pl.ds(ebase, _CH)])
        return carry

    lax.fori_loop(0, _CPW, chunk, 0)


def _sc_dd_body(row_hbm, col_hbm, x0_hbm, x1_hbm, x2_hbm, dd_hbm,
                dd_v, ir_v, ic_v, x0_v, x1_v, x2_v):
    c = lax.axis_index("c")
    s = lax.axis_index("s")
    wid = c * _NS + s

    pltpu.sync_copy(x0_hbm, x0_v)
    pltpu.sync_copy(x1_hbm, x1_v)
    pltpu.sync_copy(x2_hbm, x2_v)

    def zrow(i, carry):
        dd_v[i, pl.ds(0, 16)] = jnp.zeros((16,), F32)
        return carry

    lax.fori_loop(0, _CH, zrow, 0)

    def chunk(j, carry):
        ebase = wid * _EW + j * _CH
        pltpu.sync_copy(row_hbm.at[pl.ds(ebase, _CH)], ir_v)
        pltpu.sync_copy(col_hbm.at[pl.ds(ebase, _CH)], ic_v)
        for g in range(_CH // 16):
            o = 16 * g
            ir16 = ir_v[pl.ds(o, 16)]
            ic16 = ic_v[pl.ds(o, 16)]
            irh, irl = ir16 >> 7, ir16 & 127
            ich, icl = ic16 >> 7, ic16 & 127
            d0 = (plsc.load_gather(x0_v, [irh, irl])
                  - plsc.load_gather(x0_v, [ich, icl]))
            d1 = (plsc.load_gather(x1_v, [irh, irl])
                  - plsc.load_gather(x1_v, [ich, icl]))
            d2 = (plsc.load_gather(x2_v, [irh, irl])
                  - plsc.load_gather(x2_v, [ich, icl]))
            dist = d0 * d0 + d1 * d1 + d2 * d2
            rows = lax.iota(jnp.int32, 16) + o
            plsc.store_scatter(dd_v, [rows, jnp.full((16,), 0, jnp.int32)], d0)
            plsc.store_scatter(dd_v, [rows, jnp.full((16,), 1, jnp.int32)], d1)
            plsc.store_scatter(dd_v, [rows, jnp.full((16,), 2, jnp.int32)], d2)
            plsc.store_scatter(dd_v, [rows, jnp.full((16,), 3, jnp.int32)], dist)
        pltpu.sync_copy(dd_v, dd_hbm.at[pl.ds(ebase, _CH)])
        return carry

    lax.fori_loop(0, _CPW, chunk, 0)


def _edge_mlp_body(hr_ref, hc_ref, dd_ref, w1a_ref, w1b_ref, w1c_ref, b1_ref,
                   w2_ref, b2_ref, w5_ref, b5_ref, m_ref, tr_ref):
    dd = dd_ref[...]
    dist = dd[:, 3:4]
    lane = lax.broadcasted_iota(jnp.int32, dd.shape, 1)
    diff = jnp.where(lane < 3, dd, 0.0)
    t = (jnp.dot(hr_ref[...], w1a_ref[...], preferred_element_type=F32)
         + jnp.dot(hc_ref[...], w1b_ref[...], preferred_element_type=F32)
         + b1_ref[...])
    u = _silu(t + dist * w1c_ref[...])
    mm = _silu(jnp.dot(u, w2_ref[...], preferred_element_type=F32) + b2_ref[...])
    cw = jnp.tanh(jnp.sum(mm * w5_ref[...], axis=1, keepdims=True) + b5_ref[...])
    m_ref[...] = mm
    tr_ref[...] = diff * cw


def _make_scatter_body(width):
    def body(v_hbm, row2_hbm, parts_hbm, v_v, idx_v, sem_s, acc):
        c = lax.axis_index("c")
        s = lax.axis_index("s")
        wid = c * _NS + s

        def zrow(i, carry):
            for k in range(width // 16):
                v_v[i, pl.ds(16 * k, 16)] = jnp.zeros((16,), F32)
            return carry

        lax.fori_loop(0, _SUB, zrow, 0)
        nbase = s * _NPT
        for q in range(_NPT // _SUB):
            pltpu.sync_copy(v_v, acc.at[pl.ds(nbase + q * _SUB, _SUB)])
        plsc.subcore_barrier()

        nsup = jnp.where(wid < _NSUP_EXTRA, _NSUP_HI, _NSUP_LO)

        def chunk(i, carry):
            g = wid + _NW * i
            pltpu.sync_copy(row2_hbm.at[pl.ds(g * 8, 8)], idx_v)
            for k in range(8):
                ebase = g * _SCH + k * _SUB
                pltpu.sync_copy(v_hbm.at[pl.ds(ebase, _SUB)], v_v)
                dm = pltpu.async_copy(v_v, acc.at[idx_v.at[k]], sem_s,
                                      add=True)
                dm.wait()
            return carry

        lax.fori_loop(0, nsup, chunk, 0)
        plsc.subcore_barrier()
        pltpu.sync_copy(acc.at[pl.ds(nbase, _NPT)],
                        parts_hbm.at[c, pl.ds(nbase, _NPT)])
    return body


_sc_scatter_m_body = _make_scatter_body(128)


def _sc_scatter_x_body(v_hbm, row2_hbm, parts_hbm, v16_v, v128_v, idx_v,
                       sem_s, acc):
    c = lax.axis_index("c")
    s = lax.axis_index("s")
    wid = c * _NS + s

    def zrow(i, carry):
        for k in range(8):
            v128_v[i, pl.ds(16 * k, 16)] = jnp.zeros((16,), F32)
        return carry

    lax.fori_loop(0, _SUB, zrow, 0)
    nbase = s * _NPT
    for q in range(_NPT // _SUB):
        pltpu.sync_copy(v128_v, acc.at[pl.ds(nbase + q * _SUB, _SUB)])
    plsc.subcore_barrier()

    nsup = jnp.where(wid < _NSUP_EXTRA, _NSUP_HI, _NSUP_LO)

    def chunk(i, carry):
        g = wid + _NW * i
        pltpu.sync_copy(row2_hbm.at[pl.ds(g * 8, 8)], idx_v)
        for k in range(8):
            ebase = g * _SCH + k * _SUB
            pltpu.sync_copy(v_hbm.at[pl.ds(ebase, _SUB)], v16_v)

            def crow(r, carry2):
                v128_v[r, pl.ds(0, 16)] = v16_v[r, pl.ds(0, 16)]
                return carry2

            lax.fori_loop(0, _SUB, crow, 0)
            dm = pltpu.async_copy(v128_v, acc.at[idx_v.at[k]], sem_s,
                                  add=True)
            dm.wait()
        return carry

    lax.fori_loop(0, nsup, chunk, 0)
    plsc.subcore_barrier()
    pltpu.sync_copy(acc.at[pl.ds(nbase, _NPT)],
                    parts_hbm.at[c, pl.ds(nbase, _NPT)])


def _node_mlp_body(h_ref, mp_ref, xp_ref, xap_ref, w3_ref, b3_ref, w4_ref,
                   b4_ref, hn_ref, xn_ref):
    h = h_ref[...]
    ci = h.shape[1]
    magg = mp_ref[0] + mp_ref[1]
    w3 = w3_ref[...]
    u = (jnp.dot(h, w3[:ci], preferred_element_type=F32)
         + jnp.dot(magg, w3[ci:], preferred_element_type=F32) + b3_ref[...])
    hn_ref[...] = jnp.dot(_silu(u), w4_ref[...],
                          preferred_element_type=F32) + b4_ref[...]
    xn_ref[...] = xp_ref[...] + xap_ref[0] + xap_ref[1]


def kernel(h, x, edge_index, W1, b1, W2, b2, W3, b3, W4, b4, W5, b5):
    N, CI = h.shape
    E = edge_index.shape[1]
    CO = W2.shape[1]
    row = edge_index[0]
    col = edge_index[1]
    row2 = row.reshape(E // _SUB, _SUB)
    xp = jnp.zeros((N, 16), F32).at[:, :3].set(x)
    W1a = W1[:CI]
    W1b = W1[CI:2 * CI]
    w1c = W1[2 * CI:]
    b1r = b1.reshape(1, CO)
    b2r = b2.reshape(1, CO)
    b3r = b3.reshape(1, CO)
    b4r = b4.reshape(1, CO)
    w5r = W5.reshape(1, CO)
    b5r = b5.reshape(1, 1)

    nb = 5
    nrows = N // nb
    eb = 125
    erows = E // eb

    gather = pl.kernel(
        _sc_gather1_body,
        out_type=jax.ShapeDtypeStruct((E, CI), F32),
        mesh=plsc.VectorSubcoreMesh(core_axis_name="c", subcore_axis_name="s"),
        scratch_types=[pltpu.VMEM((_CH, CI), F32),
                       pltpu.VMEM((_CH,), jnp.int32),
                       pltpu.SemaphoreType.DMA],
        compiler_params=pltpu.CompilerParams(needs_layout_passes=False),
    )
    hr = gather(h, row)
    hc = gather(h, col)

    ddk = pl.kernel(
        _sc_dd_body,
        out_type=jax.ShapeDtypeStruct((E, 16), F32),
        mesh=plsc.VectorSubcoreMesh(core_axis_name="c", subcore_axis_name="s"),
        scratch_types=[pltpu.VMEM((_CH, 16), F32),
                       pltpu.VMEM((_CH,), jnp.int32),
                       pltpu.VMEM((_CH,), jnp.int32),
                       pltpu.VMEM((_XR, 128), F32),
                       pltpu.VMEM((_XR, 128), F32),
                       pltpu.VMEM((_XR, 128), F32)],
        compiler_params=pltpu.CompilerParams(needs_layout_passes=False),
    )
    xq = [jnp.pad(x[:, i], (0, _XR * 128 - N)).reshape(_XR, 128)
          for i in range(3)]
    dd = ddk(row, col, xq[0], xq[1], xq[2])

    m, tr = pl.pallas_call(
        _edge_mlp_body,
        grid=(eb,),
        in_specs=[pl.BlockSpec((erows, CI), lambda i: (i, 0)),
                  pl.BlockSpec((erows, CI), lambda i: (i, 0)),
                  pl.BlockSpec((erows, 16), lambda i: (i, 0)),
                  pl.BlockSpec((CI, CO), lambda i: (0, 0)),
                  pl.BlockSpec((CI, CO), lambda i: (0, 0)),
                  pl.BlockSpec((1, CO), lambda i: (0, 0)),
                  pl.BlockSpec((1, CO), lambda i: (0, 0)),
                  pl.BlockSpec((CO, CO), lambda i: (0, 0)),
                  pl.BlockSpec((1, CO), lambda i: (0, 0)),
                  pl.BlockSpec((1, CO), lambda i: (0, 0)),
                  pl.BlockSpec((1, 1), lambda i: (0, 0))],
        out_specs=[pl.BlockSpec((erows, CO), lambda i: (i, 0)),
                   pl.BlockSpec((erows, 16), lambda i: (i, 0))],
        out_shape=[jax.ShapeDtypeStruct((E, CO), F32),
                   jax.ShapeDtypeStruct((E, 16), F32)],
    )(hr, hc, dd, W1a, W1b, w1c, b1r, W2, b2r, w5r, b5r)

    scat_m = pl.kernel(
        _sc_scatter_m_body,
        out_type=jax.ShapeDtypeStruct((_NC, _NPAD, CO), F32),
        mesh=plsc.VectorSubcoreMesh(core_axis_name="c", subcore_axis_name="s"),
        scratch_types=[pltpu.VMEM((_SUB, CO), F32),
                       pltpu.VMEM((8, _SUB), jnp.int32),
                       pltpu.SemaphoreType.DMA,
                       pltpu.VMEM_SHARED((_NPAD, CO), F32)],
        compiler_params=pltpu.CompilerParams(needs_layout_passes=False),
    )
    scat_x = pl.kernel(
        _sc_scatter_x_body,
        out_type=jax.ShapeDtypeStruct((_NC, _NPAD, CO), F32),
        mesh=plsc.VectorSubcoreMesh(core_axis_name="c", subcore_axis_name="s"),
        scratch_types=[pltpu.VMEM((_SUB, 16), F32),
                       pltpu.VMEM((_SUB, CO), F32),
                       pltpu.VMEM((8, _SUB), jnp.int32),
                       pltpu.SemaphoreType.DMA,
                       pltpu.VMEM_SHARED((_NPAD, CO), F32)],
        compiler_params=pltpu.CompilerParams(needs_layout_passes=False),
    )
    mparts_p = scat_m(m, row2)
    xparts_p = scat_x(tr, row2)
    mparts = mparts_p[:, :N]
    xparts = xparts_p[:, :N, :16]

    h_new, x16 = pl.pallas_call(
        _node_mlp_body,
        grid=(nb,),
        in_specs=[pl.BlockSpec((nrows, CI), lambda i: (i, 0)),
                  pl.BlockSpec((_NC, nrows, CO), lambda i: (0, i, 0)),
                  pl.BlockSpec((nrows, 16), lambda i: (i, 0)),
                  pl.BlockSpec((_NC, nrows, 16), lambda i: (0, i, 0)),
                  pl.BlockSpec((CI + CO, CO), lambda i: (0, 0)),
                  pl.BlockSpec((1, CO), lambda i: (0, 0)),
                  pl.BlockSpec((CO, CO), lambda i: (0, 0)),
                  pl.BlockSpec((1, CO), lambda i: (0, 0))],
        out_specs=[pl.BlockSpec((nrows, CO), lambda i: (i, 0)),
                   pl.BlockSpec((nrows, 16), lambda i: (i, 0))],
        out_shape=[jax.ShapeDtypeStruct((N, CO), F32),
                   jax.ShapeDtypeStruct((N, 16), F32)],
    )(h, mparts, xp, xparts, W3, b3r, W4, b4r)

    return (h_new, x16[:, :3])


# speedup vs baseline: 4.2137x; 1.0763x over previous
import jax
import jax.numpy as jnp
from jax import lax
from jax.experimental import pallas as pl
from jax.experimental.pallas import tpu as pltpu
from jax.experimental.pallas import tpu_sc as plsc

F32 = jnp.float32

_NC, _NS = 2, 16
_NW = _NC * _NS
_SUB = 80
_NSUB = 5
_CH = _SUB * _NSUB
_EW = 10000
_CPW = _EW // _CH
_NPT = 640
_NPAD = _NS * _NPT
_SCH = 8 * _SUB
_NSUP = 320000 // _SCH
_NSUP_LO = _NSUP // _NW
_NSUP_HI = _NSUP_LO + 1
_NSUP_EXTRA = _NSUP - _NSUP_LO * _NW
_XR = 80


def _silu(v):
    return v * jax.nn.sigmoid(v)


def _sc_gather1_body(tbl_hbm, idx_hbm, out_hbm, buf_v, idx_v, sem):
    c = lax.axis_index("c")
    s = lax.axis_index("s")
    wid = c * _NS + s

    def chunk(j, carry):
        ebase = wid * _EW + j * _CH
        pltpu.sync_copy(idx_hbm.at[pl.ds(ebase, _CH)], idx_v)
        da = []
        for k in range(_NSUB):
            sl = pl.ds(_SUB * k, _SUB)
            da.append(pltpu.async_copy(tbl_hbm.at[idx_v.at[sl]], buf_v.at[sl],
                                       sem))
        for d in da:
            d.wait()
        pltpu.sync_copy(buf_v, out_hbm.at[pl.ds(ebase, _CH)])
        return carry

    lax.fori_loop(0, _CPW, chunk, 0)


def _sc_dd_body(row_hbm, col_hbm, x0_hbm, x1_hbm, x2_hbm, dd_hbm,
                dd_v, ir_v, ic_v, x0_v, x1_v, x2_v):
    c = lax.axis_index("c")
    s = lax.axis_index("s")
    wid = c * _NS + s

    pltpu.sync_copy(x0_hbm, x0_v)
    pltpu.sync_copy(x1_hbm, x1_v)
    pltpu.sync_copy(x2_hbm, x2_v)

    def zrow(i, carry):
        dd_v[i, pl.ds(0, 16)] = jnp.zeros((16,), F32)
        return carry

    lax.fori_loop(0, _CH, zrow, 0)

    def chunk(j, carry):
        ebase = wid * _EW + j * _CH
        pltpu.sync_copy(row_hbm.at[pl.ds(ebase, _CH)], ir_v)
        pltpu.sync_copy(col_hbm.at[pl.ds(ebase, _CH)], ic_v)
        for g in range(_CH // 16):
            o = 16 * g
            ir16 = ir_v[pl.ds(o, 16)]
            ic16 = ic_v[pl.ds(o, 16)]
            irh, irl = ir16 >> 7, ir16 & 127
            ich, icl = ic16 >> 7, ic16 & 127
            d0 = (plsc.load_gather(x0_v, [irh, irl])
                  - plsc.load_gather(x0_v, [ich, icl]))
            d1 = (plsc.load_gather(x1_v, [irh, irl])
                  - plsc.load_gather(x1_v, [ich, icl]))
            d2 = (plsc.load_gather(x2_v, [irh, irl])
                  - plsc.load_gather(x2_v, [ich, icl]))
            dist = d0 * d0 + d1 * d1 + d2 * d2
            rows = lax.iota(jnp.int32, 16) + o
            plsc.store_scatter(dd_v, [rows, jnp.full((16,), 0, jnp.int32)], d0)
            plsc.store_scatter(dd_v, [rows, jnp.full((16,), 1, jnp.int32)], d1)
            plsc.store_scatter(dd_v, [rows, jnp.full((16,), 2, jnp.int32)], d2)
            plsc.store_scatter(dd_v, [rows, jnp.full((16,), 3, jnp.int32)], dist)
        pltpu.sync_copy(dd_v, dd_hbm.at[pl.ds(ebase, _CH)])
        return carry

    lax.fori_loop(0, _CPW, chunk, 0)


def _edge_mlp_body(hr_ref, hc_ref, dd_ref, w1a_ref, w1b_ref, w1c_ref, b1_ref,
                   w2_ref, b2_ref, w5_ref, b5_ref, m_ref, tr_ref):
    dd = dd_ref[...]
    dist = dd[:, 3:4]
    lane = lax.broadcasted_iota(jnp.int32, dd.shape, 1)
    diff = jnp.where(lane < 3, dd, 0.0)
    t = (jnp.dot(hr_ref[...], w1a_ref[...], preferred_element_type=F32)
         + jnp.dot(hc_ref[...], w1b_ref[...], preferred_element_type=F32)
         + b1_ref[...])
    u = _silu(t + dist * w1c_ref[...])
    mm = _silu(jnp.dot(u, w2_ref[...], preferred_element_type=F32) + b2_ref[...])
    cw = jnp.tanh(jnp.sum(mm * w5_ref[...], axis=1, keepdims=True) + b5_ref[...])
    m_ref[...] = mm
    tr_ref[...] = diff * cw


def _sc_scatter_m_body(v_hbm, row2_hbm, parts_hbm, v0_v, v1_v, idx_v,
                       sem_s, acc):
    c = lax.axis_index("c")
    s = lax.axis_index("s")
    wid = c * _NS + s
    bufs = (v0_v, v1_v)

    def zrow(i, carry):
        for k in range(8):
            v0_v[i, pl.ds(16 * k, 16)] = jnp.zeros((16,), F32)
        return carry

    lax.fori_loop(0, _SUB, zrow, 0)
    nbase = s * _NPT
    for q in range(_NPT // _SUB):
        pltpu.sync_copy(v0_v, acc.at[pl.ds(nbase + q * _SUB, _SUB)])
    plsc.subcore_barrier()

    nsup = jnp.where(wid < _NSUP_EXTRA, _NSUP_HI, _NSUP_LO)

    def chunk(i, carry):
        g = wid + _NW * i
        pltpu.sync_copy(row2_hbm.at[pl.ds(g * 8, 8)], idx_v)
        descs = [None] * 8
        for k in range(8):
            b = bufs[k % 2]
            if k >= 2:
                descs[k - 2].wait()
            ebase = g * _SCH + k * _SUB
            pltpu.sync_copy(v_hbm.at[pl.ds(ebase, _SUB)], b)
            descs[k] = pltpu.async_copy(b, acc.at[idx_v.at[k]], sem_s,
                                        add=True)
        descs[6].wait()
        descs[7].wait()
        return carry

    lax.fori_loop(0, nsup, chunk, 0)
    plsc.subcore_barrier()
    pltpu.sync_copy(acc.at[pl.ds(nbase, _NPT)],
                    parts_hbm.at[c, pl.ds(nbase, _NPT)])


def _sc_scatter_x_body(v_hbm, row2_hbm, parts_hbm, v16_v, w0_v, w1_v, idx_v,
                       sem_s, acc):
    c = lax.axis_index("c")
    s = lax.axis_index("s")
    wid = c * _NS + s
    bufs = (w0_v, w1_v)

    def zrow(i, carry):
        for k in range(8):
            w0_v[i, pl.ds(16 * k, 16)] = jnp.zeros((16,), F32)
            w1_v[i, pl.ds(16 * k, 16)] = jnp.zeros((16,), F32)
        return carry

    lax.fori_loop(0, _SUB, zrow, 0)
    nbase = s * _NPT
    for q in range(_NPT // _SUB):
        pltpu.sync_copy(w0_v, acc.at[pl.ds(nbase + q * _SUB, _SUB)])
    plsc.subcore_barrier()

    nsup = jnp.where(wid < _NSUP_EXTRA, _NSUP_HI, _NSUP_LO)

    def chunk(i, carry):
        g = wid + _NW * i
        pltpu.sync_copy(row2_hbm.at[pl.ds(g * 8, 8)], idx_v)
        descs = [None] * 8
        for k in range(8):
            b = bufs[k % 2]
            if k >= 2:
                descs[k - 2].wait()
            ebase = g * _SCH + k * _SUB
            pltpu.sync_copy(v_hbm.at[pl.ds(ebase, _SUB)], v16_v)

            def crow(r, carry2, _b=b):
                _b[r, pl.ds(0, 16)] = v16_v[r, pl.ds(0, 16)]
                return carry2

            lax.fori_loop(0, _SUB, crow, 0)
            descs[k] = pltpu.async_copy(b, acc.at[idx_v.at[k]], sem_s,
                                        add=True)
        descs[6].wait()
        descs[7].wait()
        return carry

    lax.fori_loop(0, nsup, chunk, 0)
    plsc.subcore_barrier()
    pltpu.sync_copy(acc.at[pl.ds(nbase, _NPT)],
                    parts_hbm.at[c, pl.ds(nbase, _NPT)])


def _node_mlp_body(h_ref, mp_ref, xp_ref, xap_ref, w3_ref, b3_ref, w4_ref,
                   b4_ref, hn_ref, xn_ref):
    h = h_ref[...]
    ci = h.shape[1]
    magg = mp_ref[0] + mp_ref[1]
    w3 = w3_ref[...]
    u = (jnp.dot(h, w3[:ci], preferred_element_type=F32)
         + jnp.dot(magg, w3[ci:], preferred_element_type=F32) + b3_ref[...])
    hn_ref[...] = jnp.dot(_silu(u), w4_ref[...],
                          preferred_element_type=F32) + b4_ref[...]
    xn_ref[...] = xp_ref[...] + xap_ref[0] + xap_ref[1]


def kernel(h, x, edge_index, W1, b1, W2, b2, W3, b3, W4, b4, W5, b5):
    N, CI = h.shape
    E = edge_index.shape[1]
    CO = W2.shape[1]
    row = edge_index[0]
    col = edge_index[1]
    row2 = row.reshape(E // _SUB, _SUB)
    xp = jnp.zeros((N, 16), F32).at[:, :3].set(x)
    W1a = W1[:CI]
    W1b = W1[CI:2 * CI]
    w1c = W1[2 * CI:]
    b1r = b1.reshape(1, CO)
    b2r = b2.reshape(1, CO)
    b3r = b3.reshape(1, CO)
    b4r = b4.reshape(1, CO)
    w5r = W5.reshape(1, CO)
    b5r = b5.reshape(1, 1)

    nb = 5
    nrows = N // nb
    eb = 125
    erows = E // eb

    gather = pl.kernel(
        _sc_gather1_body,
        out_type=jax.ShapeDtypeStruct((E, CI), F32),
        mesh=plsc.VectorSubcoreMesh(core_axis_name="c", subcore_axis_name="s"),
        scratch_types=[pltpu.VMEM((_CH, CI), F32),
                       pltpu.VMEM((_CH,), jnp.int32),
                       pltpu.SemaphoreType.DMA],
        compiler_params=pltpu.CompilerParams(needs_layout_passes=False),
    )
    hr = gather(h, row)
    hc = gather(h, col)

    ddk = pl.kernel(
        _sc_dd_body,
        out_type=jax.ShapeDtypeStruct((E, 16), F32),
        mesh=plsc.VectorSubcoreMesh(core_axis_name="c", subcore_axis_name="s"),
        scratch_types=[pltpu.VMEM((_CH, 16), F32),
                       pltpu.VMEM((_CH,), jnp.int32),
                       pltpu.VMEM((_CH,), jnp.int32),
                       pltpu.VMEM((_XR, 128), F32),
                       pltpu.VMEM((_XR, 128), F32),
                       pltpu.VMEM((_XR, 128), F32)],
        compiler_params=pltpu.CompilerParams(needs_layout_passes=False),
    )
    xq = [jnp.pad(x[:, i], (0, _XR * 128 - N)).reshape(_XR, 128)
          for i in range(3)]
    dd = ddk(row, col, xq[0], xq[1], xq[2])

    m, tr = pl.pallas_call(
        _edge_mlp_body,
        grid=(eb,),
        in_specs=[pl.BlockSpec((erows, CI), lambda i: (i, 0)),
                  pl.BlockSpec((erows, CI), lambda i: (i, 0)),
                  pl.BlockSpec((erows, 16), lambda i: (i, 0)),
                  pl.BlockSpec((CI, CO), lambda i: (0, 0)),
                  pl.BlockSpec((CI, CO), lambda i: (0, 0)),
                  pl.BlockSpec((1, CO), lambda i: (0, 0)),
                  pl.BlockSpec((1, CO), lambda i: (0, 0)),
                  pl.BlockSpec((CO, CO), lambda i: (0, 0)),
                  pl.BlockSpec((1, CO), lambda i: (0, 0)),
                  pl.BlockSpec((1, CO), lambda i: (0, 0)),
                  pl.BlockSpec((1, 1), lambda i: (0, 0))],
        out_specs=[pl.BlockSpec((erows, CO), lambda i: (i, 0)),
                   pl.BlockSpec((erows, 16), lambda i: (i, 0))],
        out_shape=[jax.ShapeDtypeStruct((E, CO), F32),
                   jax.ShapeDtypeStruct((E, 16), F32)],
    )(hr, hc, dd, W1a, W1b, w1c, b1r, W2, b2r, w5r, b5r)

    scat_m = pl.kernel(
        _sc_scatter_m_body,
        out_type=jax.ShapeDtypeStruct((_NC, _NPAD, CO), F32),
        mesh=plsc.VectorSubcoreMesh(core_axis_name="c", subcore_axis_name="s"),
        scratch_types=[pltpu.VMEM((_SUB, CO), F32),
                       pltpu.VMEM((_SUB, CO), F32),
                       pltpu.VMEM((8, _SUB), jnp.int32),
                       pltpu.SemaphoreType.DMA,
                       pltpu.VMEM_SHARED((_NPAD, CO), F32)],
        compiler_params=pltpu.CompilerParams(needs_layout_passes=False),
    )
    scat_x = pl.kernel(
        _sc_scatter_x_body,
        out_type=jax.ShapeDtypeStruct((_NC, _NPAD, CO), F32),
        mesh=plsc.VectorSubcoreMesh(core_axis_name="c", subcore_axis_name="s"),
        scratch_types=[pltpu.VMEM((_SUB, 16), F32),
                       pltpu.VMEM((_SUB, CO), F32),
                       pltpu.VMEM((_SUB, CO), F32),
                       pltpu.VMEM((8, _SUB), jnp.int32),
                       pltpu.SemaphoreType.DMA,
                       pltpu.VMEM_SHARED((_NPAD, CO), F32)],
        compiler_params=pltpu.CompilerParams(needs_layout_passes=False),
    )
    mparts_p = scat_m(m, row2)
    xparts_p = scat_x(tr, row2)
    mparts = mparts_p[:, :N]
    xparts = xparts_p[:, :N, :16]

    h_new, x16 = pl.pallas_call(
        _node_mlp_body,
        grid=(nb,),
        in_specs=[pl.BlockSpec((nrows, CI), lambda i: (i, 0)),
                  pl.BlockSpec((_NC, nrows, CO), lambda i: (0, i, 0)),
                  pl.BlockSpec((nrows, 16), lambda i: (i, 0)),
                  pl.BlockSpec((_NC, nrows, 16), lambda i: (0, i, 0)),
                  pl.BlockSpec((CI + CO, CO), lambda i: (0, 0)),
                  pl.BlockSpec((1, CO), lambda i: (0, 0)),
                  pl.BlockSpec((CO, CO), lambda i: (0, 0)),
                  pl.BlockSpec((1, CO), lambda i: (0, 0))],
        out_specs=[pl.BlockSpec((nrows, CO), lambda i: (i, 0)),
                   pl.BlockSpec((nrows, 16), lambda i: (i, 0))],
        out_shape=[jax.ShapeDtypeStruct((N, CO), F32),
                   jax.ShapeDtypeStruct((N, 16), F32)],
    )(h, mparts, xp, xparts, W3, b3r, W4, b4r)

    return (h_new, x16[:, :3])


# speedup vs baseline: 4.3707x; 1.0373x over previous
import jax
import jax.numpy as jnp
from jax import lax
from jax.experimental import pallas as pl
from jax.experimental.pallas import tpu as pltpu
from jax.experimental.pallas import tpu_sc as plsc

F32 = jnp.float32

_NC, _NS = 2, 16
_NW = _NC * _NS
_SUB = 80
_NSUB = 5
_CH = _SUB * _NSUB
_EW = 10000
_CPW = _EW // _CH
_NPT = 640
_NPAD = _NS * _NPT
_SCH = 8 * _SUB
_NSUP = 320000 // _SCH
_NSUP_LO = _NSUP // _NW
_NSUP_HI = _NSUP_LO + 1
_NSUP_EXTRA = _NSUP - _NSUP_LO * _NW
_XR = 80
_GCH = 200
_GSUB = 40
_GNSUB = _GCH // _GSUB


def _silu(v):
    return v * jax.nn.sigmoid(v)


def _sc_gather1_body(tbl_hbm, idx_hbm, out_hbm, b0_v, b1_v, i0_v, i1_v,
                     sem_g, sem_s):
    c = lax.axis_index("c")
    s = lax.axis_index("s")
    wid = c * _NS + s
    slots = ((b0_v, i0_v), (b1_v, i1_v))

    def pair(j, carry):
        descs = []
        for p in range(2):
            b, iv = slots[p]
            ebase = wid * _EW + (2 * j + p) * _GCH

            @pl.when(j > 0)
            def _drain(b=b, ebase=ebase):
                pltpu.make_async_copy(b, out_hbm.at[pl.ds(ebase, _GCH)],
                                      sem_s).wait()

            pltpu.sync_copy(idx_hbm.at[pl.ds(ebase, _GCH)], iv)
            da = []
            for k in range(_GNSUB):
                sl = pl.ds(_GSUB * k, _GSUB)
                da.append(pltpu.async_copy(tbl_hbm.at[iv.at[sl]], b.at[sl],
                                           sem_g))
            descs.append((b, ebase, da))
        for b, ebase, da in descs:
            for d in da:
                d.wait()
            pltpu.async_copy(b, out_hbm.at[pl.ds(ebase, _GCH)], sem_s)
        return carry

    lax.fori_loop(0, _EW // (2 * _GCH), pair, 0)
    for p in range(2):
        b, _ = slots[p]
        pltpu.make_async_copy(b, out_hbm.at[pl.ds(0, _GCH)], sem_s).wait()


def _sc_dd_body(row_hbm, col_hbm, x0_hbm, x1_hbm, x2_hbm, dd_hbm,
                dd_v, ir_v, ic_v, x0_v, x1_v, x2_v):
    c = lax.axis_index("c")
    s = lax.axis_index("s")
    wid = c * _NS + s

    pltpu.sync_copy(x0_hbm, x0_v)
    pltpu.sync_copy(x1_hbm, x1_v)
    pltpu.sync_copy(x2_hbm, x2_v)

    def zrow(i, carry):
        dd_v[i, pl.ds(0, 16)] = jnp.zeros((16,), F32)
        return carry

    lax.fori_loop(0, _CH, zrow, 0)

    def chunk(j, carry):
        ebase = wid * _EW + j * _CH
        pltpu.sync_copy(row_hbm.at[pl.ds(ebase, _CH)], ir_v)
        pltpu.sync_copy(col_hbm.at[pl.ds(ebase, _CH)], ic_v)
        for g in range(_CH // 16):
            o = 16 * g
            ir16 = ir_v[pl.ds(o, 16)]
            ic16 = ic_v[pl.ds(o, 16)]
            irh, irl = ir16 >> 7, ir16 & 127
            ich, icl = ic16 >> 7, ic16 & 127
            d0 = (plsc.load_gather(x0_v, [irh, irl])
                  - plsc.load_gather(x0_v, [ich, icl]))
            d1 = (plsc.load_gather(x1_v, [irh, irl])
                  - plsc.load_gather(x1_v, [ich, icl]))
            d2 = (plsc.load_gather(x2_v, [irh, irl])
                  - plsc.load_gather(x2_v, [ich, icl]))
            dist = d0 * d0 + d1 * d1 + d2 * d2
            rows = lax.iota(jnp.int32, 16) + o
            plsc.store_scatter(dd_v, [rows, jnp.full((16,), 0, jnp.int32)], d0)
            plsc.store_scatter(dd_v, [rows, jnp.full((16,), 1, jnp.int32)], d1)
            plsc.store_scatter(dd_v, [rows, jnp.full((16,), 2, jnp.int32)], d2)
            plsc.store_scatter(dd_v, [rows, jnp.full((16,), 3, jnp.int32)], dist)
        pltpu.sync_copy(dd_v, dd_hbm.at[pl.ds(ebase, _CH)])
        return carry

    lax.fori_loop(0, _CPW, chunk, 0)


def _edge_mlp_body(hr_ref, hc_ref, dd_ref, w1a_ref, w1b_ref, w1c_ref, b1_ref,
                   w2_ref, b2_ref, w5_ref, b5_ref, m_ref, tr_ref):
    dd = dd_ref[...]
    dist = dd[:, 3:4]
    lane = lax.broadcasted_iota(jnp.int32, dd.shape, 1)
    diff = jnp.where(lane < 3, dd, 0.0)
    t = (jnp.dot(hr_ref[...], w1a_ref[...], preferred_element_type=F32)
         + jnp.dot(hc_ref[...], w1b_ref[...], preferred_element_type=F32)
         + b1_ref[...])
    u = _silu(t + dist * w1c_ref[...])
    mm = _silu(jnp.dot(u, w2_ref[...], preferred_element_type=F32) + b2_ref[...])
    cw = jnp.tanh(jnp.sum(mm * w5_ref[...], axis=1, keepdims=True) + b5_ref[...])
    m_ref[...] = mm
    tr_ref[...] = diff * cw


def _sc_scatter_m_body(v_hbm, row2_hbm, parts_hbm, v0_v, v1_v, idx_v,
                       sem_s, acc):
    c = lax.axis_index("c")
    s = lax.axis_index("s")
    wid = c * _NS + s
    bufs = (v0_v, v1_v)

    def zrow(i, carry):
        for k in range(8):
            v0_v[i, pl.ds(16 * k, 16)] = jnp.zeros((16,), F32)
        return carry

    lax.fori_loop(0, _SUB, zrow, 0)
    nbase = s * _NPT
    for q in range(_NPT // _SUB):
        pltpu.sync_copy(v0_v, acc.at[pl.ds(nbase + q * _SUB, _SUB)])
    plsc.subcore_barrier()

    nsup = jnp.where(wid < _NSUP_EXTRA, _NSUP_HI, _NSUP_LO)

    def chunk(i, carry):
        g = wid + _NW * i
        pltpu.sync_copy(row2_hbm.at[pl.ds(g * 8, 8)], idx_v)
        descs = [None] * 8
        for k in range(8):
            b = bufs[k % 2]
            if k >= 2:
                descs[k - 2].wait()
            ebase = g * _SCH + k * _SUB
            pltpu.sync_copy(v_hbm.at[pl.ds(ebase, _SUB)], b)
            descs[k] = pltpu.async_copy(b, acc.at[idx_v.at[k]], sem_s,
                                        add=True)
        descs[6].wait()
        descs[7].wait()
        return carry

    lax.fori_loop(0, nsup, chunk, 0)
    plsc.subcore_barrier()
    pltpu.sync_copy(acc.at[pl.ds(nbase, _NPT)],
                    parts_hbm.at[c, pl.ds(nbase, _NPT)])


def _sc_scatter_x_body(v_hbm, row2_hbm, parts_hbm, v16_v, w0_v, w1_v, idx_v,
                       sem_s, acc):
    c = lax.axis_index("c")
    s = lax.axis_index("s")
    wid = c * _NS + s
    bufs = (w0_v, w1_v)

    def zrow(i, carry):
        for k in range(8):
            w0_v[i, pl.ds(16 * k, 16)] = jnp.zeros((16,), F32)
            w1_v[i, pl.ds(16 * k, 16)] = jnp.zeros((16,), F32)
        return carry

    lax.fori_loop(0, _SUB, zrow, 0)
    nbase = s * _NPT
    for q in range(_NPT // _SUB):
        pltpu.sync_copy(w0_v, acc.at[pl.ds(nbase + q * _SUB, _SUB)])
    plsc.subcore_barrier()

    nsup = jnp.where(wid < _NSUP_EXTRA, _NSUP_HI, _NSUP_LO)

    def chunk(i, carry):
        g = wid + _NW * i
        pltpu.sync_copy(row2_hbm.at[pl.ds(g * 8, 8)], idx_v)
        descs = [None] * 8
        for k in range(8):
            b = bufs[k % 2]
            if k >= 2:
                descs[k - 2].wait()
            ebase = g * _SCH + k * _SUB
            pltpu.sync_copy(v_hbm.at[pl.ds(ebase, _SUB)], v16_v)

            def crow(r, carry2, _b=b):
                _b[r, pl.ds(0, 16)] = v16_v[r, pl.ds(0, 16)]
                return carry2

            lax.fori_loop(0, _SUB, crow, 0)
            descs[k] = pltpu.async_copy(b, acc.at[idx_v.at[k]], sem_s,
                                        add=True)
        descs[6].wait()
        descs[7].wait()
        return carry

    lax.fori_loop(0, nsup, chunk, 0)
    plsc.subcore_barrier()
    pltpu.sync_copy(acc.at[pl.ds(nbase, _NPT)],
                    parts_hbm.at[c, pl.ds(nbase, _NPT)])


def _node_mlp_body(h_ref, mp_ref, xp_ref, xap_ref, w3_ref, b3_ref, w4_ref,
                   b4_ref, hn_ref, xn_ref):
    h = h_ref[...]
    ci = h.shape[1]
    magg = mp_ref[0] + mp_ref[1]
    w3 = w3_ref[...]
    u = (jnp.dot(h, w3[:ci], preferred_element_type=F32)
         + jnp.dot(magg, w3[ci:], preferred_element_type=F32) + b3_ref[...])
    hn_ref[...] = jnp.dot(_silu(u), w4_ref[...],
                          preferred_element_type=F32) + b4_ref[...]
    xn_ref[...] = xp_ref[...] + xap_ref[0] + xap_ref[1]


def kernel(h, x, edge_index, W1, b1, W2, b2, W3, b3, W4, b4, W5, b5):
    N, CI = h.shape
    E = edge_index.shape[1]
    CO = W2.shape[1]
    row = edge_index[0]
    col = edge_index[1]
    row2 = row.reshape(E // _SUB, _SUB)
    xp = jnp.zeros((N, 16), F32).at[:, :3].set(x)
    W1a = W1[:CI]
    W1b = W1[CI:2 * CI]
    w1c = W1[2 * CI:]
    b1r = b1.reshape(1, CO)
    b2r = b2.reshape(1, CO)
    b3r = b3.reshape(1, CO)
    b4r = b4.reshape(1, CO)
    w5r = W5.reshape(1, CO)
    b5r = b5.reshape(1, 1)

    nb = 5
    nrows = N // nb
    eb = 125
    erows = E // eb

    gather = pl.kernel(
        _sc_gather1_body,
        out_type=jax.ShapeDtypeStruct((E, CI), F32),
        mesh=plsc.VectorSubcoreMesh(core_axis_name="c", subcore_axis_name="s"),
        scratch_types=[pltpu.VMEM((_GCH, CI), F32),
                       pltpu.VMEM((_GCH, CI), F32),
                       pltpu.VMEM((_GCH,), jnp.int32),
                       pltpu.VMEM((_GCH,), jnp.int32),
                       pltpu.SemaphoreType.DMA,
                       pltpu.SemaphoreType.DMA],
        compiler_params=pltpu.CompilerParams(needs_layout_passes=False),
    )
    hr = gather(h, row)
    hc = gather(h, col)

    ddk = pl.kernel(
        _sc_dd_body,
        out_type=jax.ShapeDtypeStruct((E, 16), F32),
        mesh=plsc.VectorSubcoreMesh(core_axis_name="c", subcore_axis_name="s"),
        scratch_types=[pltpu.VMEM((_CH, 16), F32),
                       pltpu.VMEM((_CH,), jnp.int32),
                       pltpu.VMEM((_CH,), jnp.int32),
                       pltpu.VMEM((_XR, 128), F32),
                       pltpu.VMEM((_XR, 128), F32),
                       pltpu.VMEM((_XR, 128), F32)],
        compiler_params=pltpu.CompilerParams(needs_layout_passes=False),
    )
    xq = [jnp.pad(x[:, i], (0, _XR * 128 - N)).reshape(_XR, 128)
          for i in range(3)]
    dd = ddk(row, col, xq[0], xq[1], xq[2])

    m, tr = pl.pallas_call(
        _edge_mlp_body,
        grid=(eb,),
        in_specs=[pl.BlockSpec((erows, CI), lambda i: (i, 0)),
                  pl.BlockSpec((erows, CI), lambda i: (i, 0)),
                  pl.BlockSpec((erows, 16), lambda i: (i, 0)),
                  pl.BlockSpec((CI, CO), lambda i: (0, 0)),
                  pl.BlockSpec((CI, CO), lambda i: (0, 0)),
                  pl.BlockSpec((1, CO), lambda i: (0, 0)),
                  pl.BlockSpec((1, CO), lambda i: (0, 0)),
                  pl.BlockSpec((CO, CO), lambda i: (0, 0)),
                  pl.BlockSpec((1, CO), lambda i: (0, 0)),
                  pl.BlockSpec((1, CO), lambda i: (0, 0)),
                  pl.BlockSpec((1, 1), lambda i: (0, 0))],
        out_specs=[pl.BlockSpec((erows, CO), lambda i: (i, 0)),
                   pl.BlockSpec((erows, 16), lambda i: (i, 0))],
        out_shape=[jax.ShapeDtypeStruct((E, CO), F32),
                   jax.ShapeDtypeStruct((E, 16), F32)],
    )(hr, hc, dd, W1a, W1b, w1c, b1r, W2, b2r, w5r, b5r)

    scat_m = pl.kernel(
        _sc_scatter_m_body,
        out_type=jax.ShapeDtypeStruct((_NC, _NPAD, CO), F32),
        mesh=plsc.VectorSubcoreMesh(core_axis_name="c", subcore_axis_name="s"),
        scratch_types=[pltpu.VMEM((_SUB, CO), F32),
                       pltpu.VMEM((_SUB, CO), F32),
                       pltpu.VMEM((8, _SUB), jnp.int32),
                       pltpu.SemaphoreType.DMA,
                       pltpu.VMEM_SHARED((_NPAD, CO), F32)],
        compiler_params=pltpu.CompilerParams(needs_layout_passes=False),
    )
    scat_x = pl.kernel(
        _sc_scatter_x_body,
        out_type=jax.ShapeDtypeStruct((_NC, _NPAD, CO), F32),
        mesh=plsc.VectorSubcoreMesh(core_axis_name="c", subcore_axis_name="s"),
        scratch_types=[pltpu.VMEM((_SUB, 16), F32),
                       pltpu.VMEM((_SUB, CO), F32),
                       pltpu.VMEM((_SUB, CO), F32),
                       pltpu.VMEM((8, _SUB), jnp.int32),
                       pltpu.SemaphoreType.DMA,
                       pltpu.VMEM_SHARED((_NPAD, CO), F32)],
        compiler_params=pltpu.CompilerParams(needs_layout_passes=False),
    )
    mparts_p = scat_m(m, row2)
    xparts_p = scat_x(tr, row2)
    mparts = mparts_p[:, :N]
    xparts = xparts_p[:, :N, :16]

    h_new, x16 = pl.pallas_call(
        _node_mlp_body,
        grid=(nb,),
        in_specs=[pl.BlockSpec((nrows, CI), lambda i: (i, 0)),
                  pl.BlockSpec((_NC, nrows, CO), lambda i: (0, i, 0)),
                  pl.BlockSpec((nrows, 16), lambda i: (i, 0)),
                  pl.BlockSpec((_NC, nrows, 16), lambda i: (0, i, 0)),
                  pl.BlockSpec((CI + CO, CO), lambda i: (0, 0)),
                  pl.BlockSpec((1, CO), lambda i: (0, 0)),
                  pl.BlockSpec((CO, CO), lambda i: (0, 0)),
                  pl.BlockSpec((1, CO), lambda i: (0, 0))],
        out_specs=[pl.BlockSpec((nrows, CO), lambda i: (i, 0)),
                   pl.BlockSpec((nrows, 16), lambda i: (i, 0))],
        out_shape=[jax.ShapeDtypeStruct((N, CO), F32),
                   jax.ShapeDtypeStruct((N, 16), F32)],
    )(h, mparts, xp, xparts, W3, b3r, W4, b4r)

    return (h_new, x16[:, :3])
